# baseline (speedup 1.0000x reference)
import jax
import jax.numpy as jnp
from jax.experimental import pallas as pl

_K = 1024
_D = 256
_RT = 256
_N = 16384


def _vq_block(x3_ref, w_ref, w2c_ref, idx_ref, enc_ref, q_ref, loss_ref):
    i = pl.program_id(0)
    nsub = x3_ref.shape[2] // _RT
    xb = x3_ref[0, :, pl.ds((i % nsub) * _RT, _RT)]
    s = jax.lax.dot_general(w_ref[...], xb, (((1,), (0,)), ((), ())),
                            preferred_element_type=jnp.float32)
    x2 = jnp.sum(xb * xb, axis=0, keepdims=True)
    dist = (x2 + w2c_ref[...]) - 2.0 * s
    m = jnp.min(dist, axis=0, keepdims=True)
    iota_k = jax.lax.broadcasted_iota(jnp.int32, (_K, _RT), 0)
    idxv = jnp.min(jnp.where(dist == m, iota_k, _K), axis=0)
    idx_ref[...] = idxv[None, None, :]

    idxc = idxv[:, None]
    iota_r = jax.lax.broadcasted_iota(jnp.int32, (_RT, _K), 1)
    enc = (iota_r == idxc).astype(jnp.float32)
    enc_ref[...] = enc
    q_ref[...] = jnp.dot(enc, w_ref[...],
                         preferred_element_type=jnp.float32)

    part = jnp.sum(m).reshape(1, 1)

    @pl.when(i == 0)
    def _init():
        loss_ref[...] = jnp.zeros((1, 1), jnp.float32)

    loss_ref[...] += part


def kernel(x, W):
    B, C, H, Wd = x.shape
    x3 = x.reshape(B, C, H * Wd)
    n = B * H * Wd
    w2c = jnp.sum(W ** 2, axis=1)[:, None]
    nt = n // _RT
    rpb = H * Wd // _RT

    idx3, enc, q, loss_sum = pl.pallas_call(
        _vq_block,
        grid=(nt,),
        in_specs=[
            pl.BlockSpec((1, _D, H * Wd), lambda i: (i // rpb, 0, 0)),
            pl.BlockSpec((_K, _D), lambda i: (0, 0)),
            pl.BlockSpec((_K, 1), lambda i: (0, 0)),
        ],
        out_specs=[
            pl.BlockSpec((1, 1, _RT), lambda i: (i, 0, 0)),
            pl.BlockSpec((_RT, _K), lambda i: (i, 0)),
            pl.BlockSpec((_RT, _D), lambda i: (i, 0)),
            pl.BlockSpec((1, 1), lambda i: (0, 0)),
        ],
        out_shape=[
            jax.ShapeDtypeStruct((nt, 1, _RT), jnp.int32),
            jax.ShapeDtypeStruct((n, _K), jnp.float32),
            jax.ShapeDtypeStruct((n, _D), jnp.float32),
            jax.ShapeDtypeStruct((1, 1), jnp.float32),
        ],
    )(x3, W, w2c)

    loss = 2.0 * loss_sum[0, 0] / (n * _D)
    out = jnp.transpose(q.reshape(B, H, Wd, C), (0, 3, 1, 2))
    return (loss, out, enc, idx3.reshape(-1))


# speedup vs baseline: 1.0404x; 1.0404x over previous
import jax
import jax.numpy as jnp
from jax.experimental import pallas as pl

_K = 1024
_D = 256
_RT = 256
_N = 16384


def _vq_block(x3_ref, w_ref, w2h_ref, idx_ref, enc_ref, q_ref, loss_ref):
    i = pl.program_id(0)
    xb = x3_ref[0]
    s = jax.lax.dot_general(w_ref[...], xb, (((1,), (0,)), ((), ())),
                            preferred_element_type=jnp.float32)
    x2h = 0.5 * jnp.sum(xb * xb, axis=0, keepdims=True)
    dist = (x2h + w2h_ref[...]) - s
    m = jnp.min(dist, axis=0, keepdims=True)
    iota_k = jax.lax.broadcasted_iota(jnp.int32, (_K, _RT), 0)
    idxv = jnp.min(jnp.where(dist == m, iota_k, _K), axis=0)
    idx_ref[...] = idxv[None, None, :]

    idxc = idxv[:, None]
    iota_r = jax.lax.broadcasted_iota(jnp.int32, (_RT, _K), 1)
    enc = (iota_r == idxc).astype(jnp.float32)
    enc_ref[...] = enc
    q_ref[...] = jnp.dot(enc, w_ref[...],
                         preferred_element_type=jnp.float32)

    part = 2.0 * jnp.sum(m).reshape(1, 1)

    @pl.when(i == 0)
    def _init():
        loss_ref[...] = jnp.zeros((1, 1), jnp.float32)

    loss_ref[...] += part


def kernel(x, W):
    B, C, H, Wd = x.shape
    x3 = x.reshape(B, C, H * Wd)
    n = B * H * Wd
    w2h = 0.5 * jnp.sum(W ** 2, axis=1)[:, None]
    nt = n // _RT
    rpb = H * Wd // _RT

    idx3, enc, q, loss_sum = pl.pallas_call(
        _vq_block,
        grid=(nt,),
        in_specs=[
            pl.BlockSpec((1, _D, _RT), lambda i: (i // rpb, 0, i % rpb)),
            pl.BlockSpec((_K, _D), lambda i: (0, 0)),
            pl.BlockSpec((_K, 1), lambda i: (0, 0)),
        ],
        out_specs=[
            pl.BlockSpec((1, 1, _RT), lambda i: (i, 0, 0)),
            pl.BlockSpec((_RT, _K), lambda i: (i, 0)),
            pl.BlockSpec((_RT, _D), lambda i: (i, 0)),
            pl.BlockSpec((1, 1), lambda i: (0, 0)),
        ],
        out_shape=[
            jax.ShapeDtypeStruct((nt, 1, _RT), jnp.int32),
            jax.ShapeDtypeStruct((n, _K), jnp.float32),
            jax.ShapeDtypeStruct((n, _D), jnp.float32),
            jax.ShapeDtypeStruct((1, 1), jnp.float32),
        ],
    )(x3, W, w2h)

    loss = 2.0 * loss_sum[0, 0] / (n * _D)
    out = jnp.transpose(q.reshape(B, H, Wd, C), (0, 3, 1, 2))
    return (loss, out, enc, idx3.reshape(-1))


# speedup vs baseline: 1.4451x; 1.3890x over previous
import jax
import jax.numpy as jnp
from jax.experimental import pallas as pl

_K = 1024
_D = 256
_RT = 1024
_N = 16384


def _vq_block(x3_ref, w_ref, w2h_ref, idx_ref, enc_ref, q_ref, loss_ref):
    i = pl.program_id(0)
    xb = x3_ref[0]
    s = jax.lax.dot_general(w_ref[...], xb, (((1,), (0,)), ((), ())),
                            preferred_element_type=jnp.float32)
    x2h = 0.5 * jnp.sum(xb * xb, axis=0, keepdims=True)
    dist = (x2h + w2h_ref[...]) - s
    m = jnp.min(dist, axis=0, keepdims=True)
    iota_k = jax.lax.broadcasted_iota(jnp.int32, (_K, _RT), 0)
    idxv = jnp.min(jnp.where(dist == m, iota_k, _K), axis=0)
    idx_ref[...] = idxv[None, None, :]

    idxc = idxv[:, None]
    iota_r = jax.lax.broadcasted_iota(jnp.int32, (_RT, _K), 1)
    enc = (iota_r == idxc).astype(jnp.float32)
    enc_ref[...] = enc
    q_ref[...] = jnp.dot(enc, w_ref[...],
                         preferred_element_type=jnp.float32)

    part = 2.0 * jnp.sum(m).reshape(1, 1)

    @pl.when(i == 0)
    def _init():
        loss_ref[...] = jnp.zeros((1, 1), jnp.float32)

    loss_ref[...] += part


def kernel(x, W):
    B, C, H, Wd = x.shape
    x3 = x.reshape(B, C, H * Wd)
    n = B * H * Wd
    w2h = 0.5 * jnp.sum(W ** 2, axis=1)[:, None]
    nt = n // _RT
    rpb = H * Wd // _RT

    idx3, enc, q, loss_sum = pl.pallas_call(
        _vq_block,
        grid=(nt,),
        in_specs=[
            pl.BlockSpec((1, _D, _RT), lambda i: (i // rpb, 0, i % rpb)),
            pl.BlockSpec((_K, _D), lambda i: (0, 0)),
            pl.BlockSpec((_K, 1), lambda i: (0, 0)),
        ],
        out_specs=[
            pl.BlockSpec((1, 1, _RT), lambda i: (i, 0, 0)),
            pl.BlockSpec((_RT, _K), lambda i: (i, 0)),
            pl.BlockSpec((_RT, _D), lambda i: (i, 0)),
            pl.BlockSpec((1, 1), lambda i: (0, 0)),
        ],
        out_shape=[
            jax.ShapeDtypeStruct((nt, 1, _RT), jnp.int32),
            jax.ShapeDtypeStruct((n, _K), jnp.float32),
            jax.ShapeDtypeStruct((n, _D), jnp.float32),
            jax.ShapeDtypeStruct((1, 1), jnp.float32),
        ],
    )(x3, W, w2h)

    loss = 2.0 * loss_sum[0, 0] / (n * _D)
    out = jnp.transpose(q.reshape(B, H, Wd, C), (0, 3, 1, 2))
    return (loss, out, enc, idx3.reshape(-1))
